# Initial kernel scaffold; baseline (speedup 1.0000x reference)
import jax
import jax.numpy as jnp
from jax.experimental import pallas as pl


def kernel(x, edge_index, W1, b1, W2, b2, W3, b3, Wout, bout):
    raise NotImplementedError("write your pallas kernel here")


# speedup vs baseline: 5.5730x; 5.5730x over previous
import functools

import jax
import jax.numpy as jnp
from jax import lax
from jax.experimental import pallas as pl
from jax.experimental.pallas import tpu as pltpu
from jax.experimental.pallas import tpu_sc as plsc

N = 10000
NP = 10240
E = 320000
D = 128
HALF = 64
NS = 16
RPS = NP // NS
CH = 80
NCH = E // CH
CPS = NCH // NS
ZR = 128
RB = 1000

_f32 = jnp.float32
_mesh = plsc.VectorSubcoreMesh(core_axis_name="c", subcore_axis_name="s")
_sc_params = pltpu.CompilerParams(use_tc_tiling_on_sc=False)


@functools.partial(
    pl.kernel,
    out_type=(jax.ShapeDtypeStruct((NP, 16), _f32),
              jax.ShapeDtypeStruct((NP, 16), _f32)),
    mesh=_mesh,
    scratch_types=[
        pltpu.VMEM_SHARED((NP, 16), _f32),
        pltpu.VMEM((CPS, CH), jnp.int32),
        pltpu.VMEM((CH, 16), _f32),
        pltpu.VMEM((RPS, 16), _f32),
    ],
    compiler_params=_sc_params,
)
def _sc_degrees(src_hbm, dst_hbm, deg_o_hbm, deg_i_hbm,
                acc_sh, idx_v, ones_v, zero_v):
    core = lax.axis_index("c")
    sid = lax.axis_index("s")
    row0 = sid * RPS

    @pl.loop(0, CH)
    def _init_ones(r):
        ones_v[pl.ds(r, 1), :] = jnp.ones((1, 16), _f32)

    @pl.loop(0, RPS)
    def _init_zero(r):
        zero_v[pl.ds(r, 1), :] = jnp.zeros((1, 16), _f32)

    pltpu.sync_copy(zero_v, acc_sh.at[pl.ds(row0, RPS)])
    plsc.subcore_barrier()

    def _histogram(idx_hbm):
        pltpu.sync_copy(idx_hbm.at[sid], idx_v)

        @pl.loop(0, CPS)
        def _chunk(j):
            pltpu.sync_copy(ones_v, acc_sh.at[idx_v.at[j]], add=True)

    pl.when(core == 0)(lambda: _histogram(src_hbm))
    pl.when(core == 1)(lambda: _histogram(dst_hbm))
    plsc.subcore_barrier()

    pl.when(core == 0)(lambda: pltpu.sync_copy(
        acc_sh.at[pl.ds(row0, RPS)], deg_o_hbm.at[pl.ds(row0, RPS)]))
    pl.when(core == 1)(lambda: pltpu.sync_copy(
        acc_sh.at[pl.ds(row0, RPS)], deg_i_hbm.at[pl.ds(row0, RPS)]))


@functools.partial(
    pl.kernel,
    out_type=(jax.ShapeDtypeStruct((NP, HALF), _f32),
              jax.ShapeDtypeStruct((NP, HALF), _f32)),
    mesh=_mesh,
    scratch_types=[
        pltpu.VMEM_SHARED((NP, HALF), _f32),
        pltpu.VMEM((CPS, CH), jnp.int32),
        pltpu.VMEM((CPS, CH), jnp.int32),
        pltpu.VMEM((CH, HALF), _f32),
        pltpu.VMEM((ZR, HALF), _f32),
        pltpu.SemaphoreType.DMA,
    ],
    compiler_params=_sc_params,
)
def _sc_spmm(h0_hbm, h1_hbm, src_hbm, dst_hbm, a0_hbm, a1_hbm,
             acc_sh, sidx_v, didx_v, rows_v, zero_v, sem):
    core = lax.axis_index("c")
    sid = lax.axis_index("s")
    row0 = sid * RPS

    @pl.loop(0, ZR)
    def _init_zero(r):
        for c4 in range(HALF // 16):
            zero_v[pl.ds(r, 1), pl.ds(c4 * 16, 16)] = jnp.zeros((1, 16), _f32)

    @pl.loop(0, RPS // ZR)
    def _memset(k):
        pltpu.sync_copy(zero_v, acc_sh.at[pl.ds(row0 + k * ZR, ZR)])

    pltpu.sync_copy(src_hbm.at[sid], sidx_v)
    pltpu.sync_copy(dst_hbm.at[sid], didx_v)
    plsc.subcore_barrier()

    def _edges(tab_hbm):
        @pl.loop(0, CPS)
        def _chunk(j):
            pltpu.async_copy(tab_hbm.at[sidx_v.at[j]], rows_v, sem).wait()
            pltpu.sync_copy(rows_v, acc_sh.at[didx_v.at[j]], add=True)

    pl.when(core == 0)(lambda: _edges(h0_hbm))
    pl.when(core == 1)(lambda: _edges(h1_hbm))
    plsc.subcore_barrier()

    pl.when(core == 0)(lambda: pltpu.sync_copy(
        acc_sh.at[pl.ds(row0, RPS)], a0_hbm.at[pl.ds(row0, RPS)]))
    pl.when(core == 1)(lambda: pltpu.sync_copy(
        acc_sh.at[pl.ds(row0, RPS)], a1_hbm.at[pl.ds(row0, RPS)]))


def _norm(deg):
    return jnp.where(deg > 0, lax.rsqrt(jnp.maximum(deg, 1.0)), 0.0)


def _prep_body(x_ref, dgo_ref, h0_ref, h1_ref):
    ns = _norm(dgo_ref[:, 0:1])
    hs = x_ref[...] * ns
    h0_ref[...] = hs[:, 0:HALF]
    h1_ref[...] = hs[:, HALF:D]


def _tc_prep(x, deg_o):
    return pl.pallas_call(
        _prep_body,
        grid=(N // RB,),
        in_specs=[pl.BlockSpec((RB, D), lambda i: (i, 0)),
                  pl.BlockSpec((RB, 16), lambda i: (i, 0))],
        out_specs=[pl.BlockSpec((RB, HALF), lambda i: (i, 0)),
                   pl.BlockSpec((RB, HALF), lambda i: (i, 0))],
        out_shape=[jax.ShapeDtypeStruct((N, HALF), _f32)] * 2,
    )(x, deg_o)


def _layer_body(a0_ref, a1_ref, dgi_ref, dgo_ref, w_ref, b_ref,
                h0_ref, h1_ref):
    nd = _norm(dgi_ref[:, 0:1])
    h = jnp.dot(a0_ref[...] * nd, w_ref[0:HALF, :],
                preferred_element_type=_f32)
    h += jnp.dot(a1_ref[...] * nd, w_ref[HALF:D, :],
                 preferred_element_type=_f32)
    h = jnp.maximum(h + b_ref[...], 0.0)
    hs = h * _norm(dgo_ref[:, 0:1])
    h0_ref[...] = hs[:, 0:HALF]
    h1_ref[...] = hs[:, HALF:D]


def _tc_layer(a0, a1, deg_i, deg_o, w, b):
    return pl.pallas_call(
        _layer_body,
        grid=(N // RB,),
        in_specs=[pl.BlockSpec((RB, HALF), lambda i: (i, 0)),
                  pl.BlockSpec((RB, HALF), lambda i: (i, 0)),
                  pl.BlockSpec((RB, 16), lambda i: (i, 0)),
                  pl.BlockSpec((RB, 16), lambda i: (i, 0)),
                  pl.BlockSpec((D, D), lambda i: (0, 0)),
                  pl.BlockSpec((1, D), lambda i: (0, 0))],
        out_specs=[pl.BlockSpec((RB, HALF), lambda i: (i, 0)),
                   pl.BlockSpec((RB, HALF), lambda i: (i, 0))],
        out_shape=[jax.ShapeDtypeStruct((N, HALF), _f32)] * 2,
    )(a0, a1, deg_i, deg_o, w, b)


def _final_body(a0_ref, a1_ref, dgi_ref, w_ref, b_ref, wo_ref, bo_ref,
                out_ref, acc_ref):
    i = pl.program_id(0)
    nd = _norm(dgi_ref[:, 0:1])
    h = jnp.dot(a0_ref[...] * nd, w_ref[0:HALF, :],
                preferred_element_type=_f32)
    h += jnp.dot(a1_ref[...] * nd, w_ref[HALF:D, :],
                 preferred_element_type=_f32)
    h = jnp.maximum(h + b_ref[...], 0.0)
    part = jnp.sum(h, axis=0, keepdims=True)

    @pl.when(i == 0)
    def _():
        acc_ref[...] = jnp.zeros_like(acc_ref)

    acc_ref[...] += part

    @pl.when(i == pl.num_programs(0) - 1)
    def _():
        out_ref[...] = jnp.dot(acc_ref[...] * (1.0 / N), wo_ref[...],
                               preferred_element_type=_f32) + bo_ref[...]


def _tc_final(a0, a1, deg_i, w, b, wo, bo):
    return pl.pallas_call(
        _final_body,
        grid=(N // RB,),
        in_specs=[pl.BlockSpec((RB, HALF), lambda i: (i, 0)),
                  pl.BlockSpec((RB, HALF), lambda i: (i, 0)),
                  pl.BlockSpec((RB, 16), lambda i: (i, 0)),
                  pl.BlockSpec((D, D), lambda i: (0, 0)),
                  pl.BlockSpec((1, D), lambda i: (0, 0)),
                  pl.BlockSpec((D, D), lambda i: (0, 0)),
                  pl.BlockSpec((1, D), lambda i: (0, 0))],
        out_specs=pl.BlockSpec((1, D), lambda i: (0, 0)),
        out_shape=jax.ShapeDtypeStruct((1, D), _f32),
        scratch_shapes=[pltpu.VMEM((1, D), _f32)],
    )(a0, a1, deg_i, w, b, wo, bo)


def kernel(x, edge_index, W1, b1, W2, b2, W3, b3, Wout, bout):
    src2 = edge_index[0].reshape(NS, CPS, CH)
    dst2 = edge_index[1].reshape(NS, CPS, CH)
    deg_o, deg_i = _sc_degrees(src2, dst2)
    h0, h1 = _tc_prep(x, deg_o)
    a0, a1 = _sc_spmm(h0, h1, src2, dst2)
    h0, h1 = _tc_layer(a0, a1, deg_i, deg_o, W1, b1.reshape(1, D))
    a0, a1 = _sc_spmm(h0, h1, src2, dst2)
    h0, h1 = _tc_layer(a0, a1, deg_i, deg_o, W2, b2.reshape(1, D))
    a0, a1 = _sc_spmm(h0, h1, src2, dst2)
    return _tc_final(a0, a1, deg_i, W3, b3.reshape(1, D),
                     Wout, bout.reshape(1, D))


# speedup vs baseline: 9.2473x; 1.6593x over previous
import functools

import jax
import jax.numpy as jnp
from jax import lax
from jax.experimental import pallas as pl
from jax.experimental.pallas import tpu as pltpu
from jax.experimental.pallas import tpu_sc as plsc

N = 10000
NP = 10240
E = 320000
D = 128
HALF = 64
NS = 16
RPS = NP // NS
CH = 80
NCH = E // CH
CPS = NCH // NS
ZR = 128
RB = 1000

_f32 = jnp.float32
_mesh = plsc.VectorSubcoreMesh(core_axis_name="c", subcore_axis_name="s")
_sc_params = pltpu.CompilerParams(use_tc_tiling_on_sc=False)


@functools.partial(
    pl.kernel,
    out_type=(jax.ShapeDtypeStruct((NP, 16), _f32),
              jax.ShapeDtypeStruct((NP, 16), _f32)),
    mesh=_mesh,
    scratch_types=[
        pltpu.VMEM_SHARED((NP, 16), _f32),
        pltpu.VMEM((CPS, CH), jnp.int32),
        pltpu.VMEM((CH, 16), _f32),
        pltpu.VMEM((RPS, 16), _f32),
        pltpu.SemaphoreType.DMA,
    ],
    compiler_params=_sc_params,
)
def _sc_degrees(src_hbm, dst_hbm, deg_o_hbm, deg_i_hbm,
                acc_sh, idx_v, ones_v, zero_v, sem):
    core = lax.axis_index("c")
    sid = lax.axis_index("s")
    row0 = sid * RPS

    @pl.loop(0, CH)
    def _init_ones(r):
        ones_v[pl.ds(r, 1), :] = jnp.ones((1, 16), _f32)

    @pl.loop(0, RPS)
    def _init_zero(r):
        zero_v[pl.ds(r, 1), :] = jnp.zeros((1, 16), _f32)

    pltpu.sync_copy(zero_v, acc_sh.at[pl.ds(row0, RPS)])
    plsc.subcore_barrier()

    def _histogram(idx_hbm):
        pltpu.sync_copy(idx_hbm.at[sid], idx_v)

        @pl.loop(0, CPS, step=10)
        def _chunk(j):
            for k in range(10):
                pltpu.async_copy(ones_v, acc_sh.at[idx_v.at[j + k]], sem,
                                 add=True)
            for k in range(10):
                pltpu.make_async_copy(ones_v, acc_sh.at[idx_v.at[j + k]],
                                      sem).wait()

    pl.when(core == 0)(lambda: _histogram(src_hbm))
    pl.when(core == 1)(lambda: _histogram(dst_hbm))
    plsc.subcore_barrier()

    pl.when(core == 0)(lambda: pltpu.sync_copy(
        acc_sh.at[pl.ds(row0, RPS)], deg_o_hbm.at[pl.ds(row0, RPS)]))
    pl.when(core == 1)(lambda: pltpu.sync_copy(
        acc_sh.at[pl.ds(row0, RPS)], deg_i_hbm.at[pl.ds(row0, RPS)]))


@functools.partial(
    pl.kernel,
    out_type=(jax.ShapeDtypeStruct((NP, HALF), _f32),
              jax.ShapeDtypeStruct((NP, HALF), _f32)),
    mesh=_mesh,
    scratch_types=[
        pltpu.VMEM_SHARED((NP, HALF), _f32),
        pltpu.VMEM((CPS, CH), jnp.int32),
        pltpu.VMEM((CPS, CH), jnp.int32),
        pltpu.VMEM((CH, HALF), _f32),
        pltpu.VMEM((CH, HALF), _f32),
        pltpu.VMEM((ZR, HALF), _f32),
        pltpu.SemaphoreType.DMA,
        pltpu.SemaphoreType.DMA,
    ],
    compiler_params=_sc_params,
)
def _sc_spmm(h0_hbm, h1_hbm, src_hbm, dst_hbm, a0_hbm, a1_hbm,
             acc_sh, sidx_v, didx_v, rows_a, rows_b, zero_v, sem_a, sem_b):
    core = lax.axis_index("c")
    sid = lax.axis_index("s")
    row0 = sid * RPS

    @pl.loop(0, ZR)
    def _init_zero(r):
        for c4 in range(HALF // 16):
            zero_v[pl.ds(r, 1), pl.ds(c4 * 16, 16)] = jnp.zeros((1, 16), _f32)

    @pl.loop(0, RPS // ZR)
    def _memset(k):
        pltpu.sync_copy(zero_v, acc_sh.at[pl.ds(row0 + k * ZR, ZR)])

    pltpu.sync_copy(src_hbm.at[sid], sidx_v)
    pltpu.sync_copy(dst_hbm.at[sid], didx_v)
    plsc.subcore_barrier()

    def _edges(tab_hbm):
        pltpu.async_copy(tab_hbm.at[sidx_v.at[0]], rows_a, sem_a)

        @pl.loop(0, CPS, step=2)
        def _chunk(j):
            pltpu.async_copy(tab_hbm.at[sidx_v.at[j + 1]], rows_b, sem_b)
            pltpu.make_async_copy(tab_hbm.at[sidx_v.at[j]], rows_a,
                                  sem_a).wait()
            pltpu.sync_copy(rows_a, acc_sh.at[didx_v.at[j]], add=True)

            @pl.when(j + 2 < CPS)
            def _():
                pltpu.async_copy(tab_hbm.at[sidx_v.at[j + 2]], rows_a, sem_a)

            pltpu.make_async_copy(tab_hbm.at[sidx_v.at[j + 1]], rows_b,
                                  sem_b).wait()
            pltpu.sync_copy(rows_b, acc_sh.at[didx_v.at[j + 1]], add=True)

    pl.when(core == 0)(lambda: _edges(h0_hbm))
    pl.when(core == 1)(lambda: _edges(h1_hbm))
    plsc.subcore_barrier()

    pl.when(core == 0)(lambda: pltpu.sync_copy(
        acc_sh.at[pl.ds(row0, RPS)], a0_hbm.at[pl.ds(row0, RPS)]))
    pl.when(core == 1)(lambda: pltpu.sync_copy(
        acc_sh.at[pl.ds(row0, RPS)], a1_hbm.at[pl.ds(row0, RPS)]))


def _norm(deg):
    return jnp.where(deg > 0, lax.rsqrt(jnp.maximum(deg, 1.0)), 0.0)


def _prep_body(x_ref, dgo_ref, h0_ref, h1_ref):
    ns = _norm(dgo_ref[:, 0:1])
    hs = x_ref[...] * ns
    h0_ref[...] = hs[:, 0:HALF]
    h1_ref[...] = hs[:, HALF:D]


def _tc_prep(x, deg_o):
    return pl.pallas_call(
        _prep_body,
        grid=(N // RB,),
        in_specs=[pl.BlockSpec((RB, D), lambda i: (i, 0)),
                  pl.BlockSpec((RB, 16), lambda i: (i, 0))],
        out_specs=[pl.BlockSpec((RB, HALF), lambda i: (i, 0)),
                   pl.BlockSpec((RB, HALF), lambda i: (i, 0))],
        out_shape=[jax.ShapeDtypeStruct((N, HALF), _f32)] * 2,
    )(x, deg_o)


def _layer_body(a0_ref, a1_ref, dgi_ref, dgo_ref, w_ref, b_ref,
                h0_ref, h1_ref):
    nd = _norm(dgi_ref[:, 0:1])
    h = jnp.dot(a0_ref[...] * nd, w_ref[0:HALF, :],
                preferred_element_type=_f32)
    h += jnp.dot(a1_ref[...] * nd, w_ref[HALF:D, :],
                 preferred_element_type=_f32)
    h = jnp.maximum(h + b_ref[...], 0.0)
    hs = h * _norm(dgo_ref[:, 0:1])
    h0_ref[...] = hs[:, 0:HALF]
    h1_ref[...] = hs[:, HALF:D]


def _tc_layer(a0, a1, deg_i, deg_o, w, b):
    return pl.pallas_call(
        _layer_body,
        grid=(N // RB,),
        in_specs=[pl.BlockSpec((RB, HALF), lambda i: (i, 0)),
                  pl.BlockSpec((RB, HALF), lambda i: (i, 0)),
                  pl.BlockSpec((RB, 16), lambda i: (i, 0)),
                  pl.BlockSpec((RB, 16), lambda i: (i, 0)),
                  pl.BlockSpec((D, D), lambda i: (0, 0)),
                  pl.BlockSpec((1, D), lambda i: (0, 0))],
        out_specs=[pl.BlockSpec((RB, HALF), lambda i: (i, 0)),
                   pl.BlockSpec((RB, HALF), lambda i: (i, 0))],
        out_shape=[jax.ShapeDtypeStruct((N, HALF), _f32)] * 2,
    )(a0, a1, deg_i, deg_o, w, b)


def _final_body(a0_ref, a1_ref, dgi_ref, w_ref, b_ref, wo_ref, bo_ref,
                out_ref, acc_ref):
    i = pl.program_id(0)
    nd = _norm(dgi_ref[:, 0:1])
    h = jnp.dot(a0_ref[...] * nd, w_ref[0:HALF, :],
                preferred_element_type=_f32)
    h += jnp.dot(a1_ref[...] * nd, w_ref[HALF:D, :],
                 preferred_element_type=_f32)
    h = jnp.maximum(h + b_ref[...], 0.0)
    part = jnp.sum(h, axis=0, keepdims=True)

    @pl.when(i == 0)
    def _():
        acc_ref[...] = jnp.zeros_like(acc_ref)

    acc_ref[...] += part

    @pl.when(i == pl.num_programs(0) - 1)
    def _():
        out_ref[...] = jnp.dot(acc_ref[...] * (1.0 / N), wo_ref[...],
                               preferred_element_type=_f32) + bo_ref[...]


def _tc_final(a0, a1, deg_i, w, b, wo, bo):
    return pl.pallas_call(
        _final_body,
        grid=(N // RB,),
        in_specs=[pl.BlockSpec((RB, HALF), lambda i: (i, 0)),
                  pl.BlockSpec((RB, HALF), lambda i: (i, 0)),
                  pl.BlockSpec((RB, 16), lambda i: (i, 0)),
                  pl.BlockSpec((D, D), lambda i: (0, 0)),
                  pl.BlockSpec((1, D), lambda i: (0, 0)),
                  pl.BlockSpec((D, D), lambda i: (0, 0)),
                  pl.BlockSpec((1, D), lambda i: (0, 0))],
        out_specs=pl.BlockSpec((1, D), lambda i: (0, 0)),
        out_shape=jax.ShapeDtypeStruct((1, D), _f32),
        scratch_shapes=[pltpu.VMEM((1, D), _f32)],
    )(a0, a1, deg_i, w, b, wo, bo)


def kernel(x, edge_index, W1, b1, W2, b2, W3, b3, Wout, bout):
    src2 = edge_index[0].reshape(NS, CPS, CH)
    dst2 = edge_index[1].reshape(NS, CPS, CH)
    deg_o, deg_i = _sc_degrees(src2, dst2)
    h0, h1 = _tc_prep(x, deg_o)
    a0, a1 = _sc_spmm(h0, h1, src2, dst2)
    h0, h1 = _tc_layer(a0, a1, deg_i, deg_o, W1, b1.reshape(1, D))
    a0, a1 = _sc_spmm(h0, h1, src2, dst2)
    h0, h1 = _tc_layer(a0, a1, deg_i, deg_o, W2, b2.reshape(1, D))
    a0, a1 = _sc_spmm(h0, h1, src2, dst2)
    return _tc_final(a0, a1, deg_i, W3, b3.reshape(1, D),
                     Wout, bout.reshape(1, D))
